# baseline (speedup 1.0000x reference)
import functools

import jax
import jax.numpy as jnp
from jax import lax
from jax.experimental import pallas as pl
from jax.experimental.pallas import tpu as pltpu
from jax.experimental.pallas import tpu_sc as plsc

_NUM_CAMS = 1_000_000
_BATCH = 16384
_NW = 32
_CPW = 32768
_CHUNK = 2048
_NCH = _CPW // _CHUNK
_CLAMP = 998016
_MAXM = _BATCH
_BIG = 2**30


@functools.cache
def _build_sc_gather():
    @functools.partial(
        pl.kernel,
        mesh=plsc.VectorSubcoreMesh(core_axis_name="c", subcore_axis_name="s"),
        out_type=jax.ShapeDtypeStruct((_BATCH, 128), jnp.float32),
        scratch_types=[
            pltpu.VMEM((_BATCH,), jnp.int32),
            pltpu.VMEM((_MAXM + 16,), jnp.int32),
            pltpu.VMEM((_MAXM + 16,), jnp.int32),
            pltpu.VMEM((2, 4, 4, _CHUNK), jnp.float32),
            pltpu.VMEM((16, 128), jnp.float32),
            pltpu.VMEM((16,), jnp.int32),
            pltpu.SemaphoreType.DMA,
            pltpu.SemaphoreType.DMA,
            pltpu.SemaphoreType.DMA,
        ],
        compiler_params=pltpu.CompilerParams(
            use_tc_tiling_on_sc=True, needs_layout_passes=False
        ),
    )
    def _sc_gather(
        pt_hbm, idx_hbm, out_hbm,
        idx_v, cl_v, bl_v, chunk_v, staged_v, bbuf_v,
        sem_a, sem_b, sem_o,
    ):
        wid = lax.axis_index("s") * 2 + lax.axis_index("c")
        clo = lax.mul(wid, _CPW)
        chi = lax.min(clo + _CPW, _NUM_CAMS)
        lane = lax.broadcasted_iota(jnp.int32, (16,), 0)
        r1_idx = lax.div(lane, 4)
        r2_idx = lax.rem(lane, 4)

        pltpu.sync_copy(idx_hbm, idx_v)

        def scan_body(g, m):
            v = idx_v[pl.ds(lax.mul(g, 16), 16)]
            msk = jnp.logical_and(v >= clo, v < chi)
            plsc.store_compressed(cl_v.at[pl.ds(m, 16)], v, mask=msk)
            plsc.store_compressed(
                bl_v.at[pl.ds(m, 16)], lax.mul(g, 16) + lane, mask=msk
            )
            return m + lax.reduce_max(
                plsc.all_reduce_population_count(msk), axes=(0,)
            )

        m = lax.fori_loop(0, _BATCH // 16, scan_body, jnp.int32(0))
        cl_v[pl.ds(m, 16)] = jnp.broadcast_to(jnp.int32(_BIG), (16,))
        nv = lax.div(m + 15, 16)

        def flush(pos_vec_valid_fixup):
            pass

        def do_flush():
            pltpu.async_copy(staged_v, out_hbm.at[bbuf_v], sem_o).wait()

        def sems_issue(k):
            return sem_a if k % 2 == 0 else sem_b

        def c0_of(k):
            c0i = clo + k * _CHUNK
            return pl.multiple_of(lax.min(c0i, _CLAMP), 128)

        def issue(k):
            @pl.when(clo + k * _CHUNK < chi)
            def _():
                pltpu.async_copy(
                    pt_hbm.at[:, :, pl.ds(c0_of(k), _CHUNK)],
                    chunk_v.at[k % 2],
                    sems_issue(k),
                )

        def drain(k):
            @pl.when(clo + k * _CHUNK < chi)
            def _():
                pltpu.make_async_copy(
                    pt_hbm.at[:, :, pl.ds(0, _CHUNK)],
                    chunk_v.at[k % 2],
                    sems_issue(k),
                ).wait()

        issue(0)
        cnt0 = jnp.int32(0)

        def make_chunk_processor(k):
            ring = k % 2
            c0i_lo = clo + k * _CHUNK
            c0i_hi = c0i_lo + _CHUNK
            c0 = c0_of(k)

            def vec_body(vix, cnt):
                cv = cl_v[pl.ds(lax.mul(vix, 16), 16)]
                bv = bl_v[pl.ds(lax.mul(vix, 16), 16)]
                msk = jnp.logical_and(cv >= c0i_lo, cv < c0i_hi)
                mi = jnp.where(msk, jnp.int32(1), jnp.int32(0))
                nrem = lax.reduce_max(
                    plsc.all_reduce_population_count(msk), axes=(0,)
                )

                def member_cond(carry):
                    _, nr, _ = carry
                    return nr > 0

                def member_body(carry):
                    mvec, nr, cn = carry
                    mb = mvec > 0
                    l = plsc.all_reduce_ffs(mb)
                    c = lax.reduce_max(
                        jnp.where(lane == l, cv, 0), axes=(0,)
                    )
                    b = lax.reduce_max(
                        jnp.where(lane == l, bv, 0), axes=(0,)
                    )
                    c_loc = jnp.broadcast_to(c - c0, (16,))
                    vals = plsc.load_gather(
                        chunk_v,
                        [
                            jnp.broadcast_to(jnp.int32(ring), (16,)),
                            r1_idx,
                            r2_idx,
                            c_loc,
                        ],
                    )
                    pos = lax.rem(cn, 16)
                    staged_v.at[pos, pl.ds(0, 16)][...] = vals
                    plsc.store_scatter(
                        bbuf_v,
                        [jnp.broadcast_to(pos, (16,))],
                        jnp.broadcast_to(b, (16,)),
                        mask=lane == 0,
                    )
                    cn = cn + 1

                    @pl.when(lax.rem(cn, 16) == 0)
                    def _():
                        do_flush()

                    mvec = jnp.where(lane == l, jnp.int32(0), mvec)
                    return (mvec, nr - 1, cn)

                _, _, cnt = lax.while_loop(
                    member_cond, member_body, (mi, nrem, cnt)
                )
                return cnt

            return vec_body

        cnt = cnt0
        for k in range(_NCH):
            if k + 1 < _NCH:
                issue(k + 1)
            drain(k)
            cnt = lax.fori_loop(0, nv, make_chunk_processor(k), cnt)

        rem = lax.rem(cnt, 16)

        @pl.when(rem > 0)
        def _():
            bvec = bbuf_v[...]
            valid = lane < rem
            bfirst = lax.reduce_max(
                jnp.where(lane == 0, bvec, 0), axes=(0,)
            )
            bbuf_v[...] = jnp.where(valid, bvec, bfirst)
            sel = jnp.where(valid, lane, 0)
            for j in range(16):
                col = plsc.load_gather(
                    staged_v, [sel, jnp.broadcast_to(jnp.int32(j), (16,))]
                )
                plsc.store_scatter(
                    staged_v,
                    [lane, jnp.broadcast_to(jnp.int32(j), (16,))],
                    col,
                )
            do_flush()

    return _sc_gather


def kernel(i, param):
    pt = jnp.transpose(param, (1, 2, 0))
    out = _build_sc_gather()(pt, i.astype(jnp.int32))
    return out[:, :16].reshape(_BATCH, 4, 4)


# speedup vs baseline: 1.1179x; 1.1179x over previous
import functools

import jax
import jax.numpy as jnp
from jax import lax
from jax.experimental import pallas as pl
from jax.experimental.pallas import tpu as pltpu
from jax.experimental.pallas import tpu_sc as plsc

_NUM_CAMS = 1_000_000
_BATCH = 16384
_NW = 32
_CPW = 32768
_CHUNK = 2048
_NCH = _CPW // _CHUNK
_CLAMP = 998016
_MAXM = _BATCH
_BIG = 2**30


@functools.cache
def _build_sc_gather():
    @functools.partial(
        pl.kernel,
        mesh=plsc.VectorSubcoreMesh(core_axis_name="c", subcore_axis_name="s"),
        out_type=jax.ShapeDtypeStruct((_BATCH, 128), jnp.float32),
        scratch_types=[
            pltpu.VMEM((_BATCH,), jnp.int32),
            pltpu.VMEM((_MAXM + 16,), jnp.int32),
            pltpu.VMEM((_MAXM + 16,), jnp.int32),
            pltpu.VMEM((2, 4, 4, _CHUNK), jnp.float32),
            pltpu.VMEM((16, 128), jnp.float32),
            pltpu.VMEM((16,), jnp.int32),
            pltpu.SemaphoreType.DMA,
            pltpu.SemaphoreType.DMA,
            pltpu.SemaphoreType.DMA,
        ],
        compiler_params=pltpu.CompilerParams(
            use_tc_tiling_on_sc=True, needs_layout_passes=False
        ),
    )
    def _sc_gather(
        pt_hbm, idx_hbm, out_hbm,
        idx_v, cl_v, bl_v, chunk_v, staged_v, bbuf_v,
        sem_a, sem_b, sem_o,
    ):
        wid = lax.axis_index("s") * 2 + lax.axis_index("c")
        clo = lax.mul(wid, _CPW)
        chi = lax.min(clo + _CPW, _NUM_CAMS)
        lane = lax.broadcasted_iota(jnp.int32, (16,), 0)
        r1_idx = lax.div(lane, 4)
        r2_idx = lax.rem(lane, 4)

        def c0_of(k):
            c0i = clo + k * _CHUNK
            return pl.multiple_of(lax.min(c0i, _CLAMP), 128)

        def sems_issue(k):
            return sem_a if k % 2 == 0 else sem_b

        def issue(k):
            @pl.when(clo + k * _CHUNK < chi)
            def _():
                pltpu.async_copy(
                    pt_hbm.at[:, :, pl.ds(c0_of(k), _CHUNK)],
                    chunk_v.at[k % 2],
                    sems_issue(k),
                )

        def drain(k):
            @pl.when(clo + k * _CHUNK < chi)
            def _():
                pltpu.make_async_copy(
                    pt_hbm.at[:, :, pl.ds(0, _CHUNK)],
                    chunk_v.at[k % 2],
                    sems_issue(k),
                ).wait()

        pltpu.sync_copy(idx_hbm, idx_v)
        issue(0)
        issue(1)

        def scan_body(g8, m):
            vs, msks, cnts = [], [], []
            for u in range(8):
                g0 = lax.mul(g8, 128) + u * 16
                v = idx_v[pl.ds(g0, 16)]
                msk = jnp.logical_and(v >= clo, v < chi)
                vs.append(v)
                msks.append(msk)
                cnts.append(
                    lax.reduce_max(
                        plsc.all_reduce_population_count(msk), axes=(0,)
                    )
                )
            for u in range(8):
                plsc.store_compressed(
                    cl_v.at[pl.ds(m, 16)], vs[u], mask=msks[u]
                )
                plsc.store_compressed(
                    bl_v.at[pl.ds(m, 16)],
                    lax.mul(g8, 128) + u * 16 + lane,
                    mask=msks[u],
                )
                m = m + cnts[u]
            return m

        m = lax.fori_loop(0, _BATCH // 128, scan_body, jnp.int32(0))
        cl_v[pl.ds(m, 16)] = jnp.broadcast_to(jnp.int32(_BIG), (16,))
        nv = lax.div(m + 15, 16)

        def flush(pos_vec_valid_fixup):
            pass

        def do_flush():
            pltpu.async_copy(staged_v, out_hbm.at[bbuf_v], sem_o).wait()

        cnt0 = jnp.int32(0)

        def make_chunk_processor(k):
            ring = k % 2
            c0i_lo = clo + k * _CHUNK
            c0i_hi = c0i_lo + _CHUNK
            c0 = c0_of(k)

            def vec_body(vix, cnt):
                cv = cl_v[pl.ds(lax.mul(vix, 16), 16)]
                bv = bl_v[pl.ds(lax.mul(vix, 16), 16)]
                msk = jnp.logical_and(cv >= c0i_lo, cv < c0i_hi)
                mi = jnp.where(msk, jnp.int32(1), jnp.int32(0))
                nrem = lax.reduce_max(
                    plsc.all_reduce_population_count(msk), axes=(0,)
                )

                def member_cond(carry):
                    _, nr, _ = carry
                    return nr > 0

                def member_body(carry):
                    mvec, nr, cn = carry
                    mb = mvec > 0
                    l = plsc.all_reduce_ffs(mb)
                    c = lax.reduce_max(
                        jnp.where(lane == l, cv, 0), axes=(0,)
                    )
                    b = lax.reduce_max(
                        jnp.where(lane == l, bv, 0), axes=(0,)
                    )
                    c_loc = jnp.broadcast_to(c - c0, (16,))
                    vals = plsc.load_gather(
                        chunk_v,
                        [
                            jnp.broadcast_to(jnp.int32(ring), (16,)),
                            r1_idx,
                            r2_idx,
                            c_loc,
                        ],
                    )
                    pos = lax.rem(cn, 16)
                    staged_v.at[pos, pl.ds(0, 16)][...] = vals
                    plsc.store_scatter(
                        bbuf_v,
                        [jnp.broadcast_to(pos, (16,))],
                        jnp.broadcast_to(b, (16,)),
                        mask=lane == 0,
                    )
                    cn = cn + 1

                    @pl.when(lax.rem(cn, 16) == 0)
                    def _():
                        do_flush()

                    mvec = jnp.where(lane == l, jnp.int32(0), mvec)
                    return (mvec, nr - 1, cn)

                _, _, cnt = lax.while_loop(
                    member_cond, member_body, (mi, nrem, cnt)
                )
                return cnt

            return vec_body

        cnt = cnt0
        for k in range(_NCH):
            drain(k)
            cnt = lax.fori_loop(0, nv, make_chunk_processor(k), cnt)
            if k + 2 < _NCH:
                issue(k + 2)

        rem = lax.rem(cnt, 16)

        @pl.when(rem > 0)
        def _():
            bvec = bbuf_v[...]
            valid = lane < rem
            bfirst = lax.reduce_max(
                jnp.where(lane == 0, bvec, 0), axes=(0,)
            )
            bbuf_v[...] = jnp.where(valid, bvec, bfirst)
            sel = jnp.where(valid, lane, 0)
            for j in range(16):
                col = plsc.load_gather(
                    staged_v, [sel, jnp.broadcast_to(jnp.int32(j), (16,))]
                )
                plsc.store_scatter(
                    staged_v,
                    [lane, jnp.broadcast_to(jnp.int32(j), (16,))],
                    col,
                )
            do_flush()

    return _sc_gather


def kernel(i, param):
    pt = jnp.transpose(param, (1, 2, 0))
    out = _build_sc_gather()(pt, i.astype(jnp.int32))
    return out[:, :16].reshape(_BATCH, 4, 4)


# speedup vs baseline: 1.1483x; 1.0272x over previous
import functools

import jax
import jax.numpy as jnp
from jax import lax
from jax.experimental import pallas as pl
from jax.experimental.pallas import tpu as pltpu
from jax.experimental.pallas import tpu_sc as plsc

_NUM_CAMS = 1_000_000
_BATCH = 16384
_NC = 2
_NS = 16
_NW = _NC * _NS
_B_PER_W = _BATCH // _NW
_RB = 16
_NR = _B_PER_W // _RB


@functools.cache
def _build_sc_gather():
    @functools.partial(
        pl.kernel,
        mesh=plsc.VectorSubcoreMesh(core_axis_name="c", subcore_axis_name="s"),
        out_type=jax.ShapeDtypeStruct((4, 4, _BATCH), jnp.float32),
        scratch_types=[
            pltpu.VMEM((_B_PER_W,), jnp.int32),
            pltpu.VMEM((2, _RB, 4, 4, 128), jnp.float32),
            pltpu.VMEM((4, 4, _B_PER_W), jnp.float32),
            pltpu.SemaphoreType.DMA,
            pltpu.SemaphoreType.DMA,
        ],
        compiler_params=pltpu.CompilerParams(
            use_tc_tiling_on_sc=True, needs_layout_passes=False
        ),
    )
    def _sc_gather(pt_hbm, idx_hbm, out_hbm, idx_v, tiles_v, buf_v, sem_a, sem_b):
        wid = lax.axis_index("s") * _NC + lax.axis_index("c")
        base = wid * _B_PER_W
        pltpu.sync_copy(idx_hbm.at[pl.ds(base, _B_PER_W)], idx_v)
        lane = lax.broadcasted_iota(jnp.int32, (16,), 0)
        r1_idx = lax.div(lane, 4)
        r2_idx = lax.rem(lane, 4)

        def issue_round(r, ring, sem):
            grp = idx_v[pl.ds(lax.mul(r, _RB), _RB)]
            col = lax.shift_left(lax.shift_right_logical(grp, 7), 7)
            for s in range(_RB):
                c0 = pl.multiple_of(
                    lax.reduce_max(jnp.where(lane == s, col, 0), axes=(0,)),
                    128,
                )
                pltpu.async_copy(
                    pt_hbm.at[:, :, pl.ds(c0, 128)],
                    tiles_v.at[ring, s],
                    sem,
                )

        def drain_round(ring, sem):
            for s in range(_RB):
                pltpu.make_async_copy(
                    pt_hbm.at[:, :, pl.ds(0, 128)],
                    tiles_v.at[ring, s],
                    sem,
                ).wait()

        def process_round(r, ring):
            grp = idx_v[pl.ds(lax.mul(r, _RB), _RB)]
            loc = lax.rem(grp, 128)
            for s in range(_RB):
                l_vec = jnp.broadcast_to(
                    lax.reduce_max(jnp.where(lane == s, loc, 0), axes=(0,)),
                    (16,),
                )
                vals = plsc.load_gather(
                    tiles_v,
                    [
                        jnp.broadcast_to(jnp.int32(ring), (16,)),
                        jnp.broadcast_to(jnp.int32(s), (16,)),
                        r1_idx,
                        r2_idx,
                        l_vec,
                    ],
                )
                j_vec = jnp.broadcast_to(lax.mul(r, _RB) + s, (16,))
                plsc.store_scatter(buf_v, [r1_idx, r2_idx, j_vec], vals)

        issue_round(0, 0, sem_a)

        def body(rr, _):
            a = lax.mul(rr, 2)
            b = a + 1
            issue_round(b, 1, sem_b)
            drain_round(0, sem_a)
            process_round(a, 0)

            @pl.when(a + 2 < _NR)
            def _():
                issue_round(a + 2, 0, sem_a)

            drain_round(1, sem_b)
            process_round(b, 1)
            return ()

        lax.fori_loop(0, _NR // 2, body, ())
        pltpu.sync_copy(buf_v, out_hbm.at[:, :, pl.ds(base, _B_PER_W)])

    return _sc_gather


def kernel(i, param):
    pt = jnp.transpose(param, (1, 2, 0))
    out_t = _build_sc_gather()(pt, i.astype(jnp.int32))
    return jnp.transpose(out_t, (2, 0, 1))
